# baseline (speedup 1.0000x reference)
import functools

import jax
import jax.numpy as jnp
from jax import lax
from jax.experimental import pallas as pl
from jax.experimental.pallas import tpu as pltpu

N_NODES = 10000
N_EDGES = 320000
LATENT = 128
BE = 2000
NEB = N_EDGES // BE
NSLOPE = 0.01


def _leaky(x):
    return jnp.where(x >= 0, x, NSLOPE * x)


def _embed_edges_body(e_ref, w_ref, b_ref, o_ref):
    o_ref[...] = jnp.dot(e_ref[...], w_ref[...],
                         preferred_element_type=jnp.float32) + b_ref[...]


def _embed_edges(edges, W, b):
    return pl.pallas_call(
        _embed_edges_body,
        grid=(NEB,),
        in_specs=[
            pl.BlockSpec((BE, 16), lambda i: (i, 0)),
            pl.BlockSpec((16, LATENT), lambda i: (0, 0)),
            pl.BlockSpec((1, LATENT), lambda i: (0, 0)),
        ],
        out_specs=pl.BlockSpec((BE, LATENT), lambda i: (i, 0)),
        out_shape=jax.ShapeDtypeStruct((N_EDGES, LATENT), jnp.float32),
    )(edges, W, b.reshape(1, LATENT))


def _embed_nodes_body(n_ref, nw_ref, nb_ref, g_ref, gw_ref, gb_ref,
                      no_ref, go_ref):
    no_ref[...] = jnp.dot(n_ref[...], nw_ref[...],
                          preferred_element_type=jnp.float32) + nb_ref[...]
    go_ref[...] = jnp.dot(g_ref[...], gw_ref[...],
                          preferred_element_type=jnp.float32) + gb_ref[...]


def _embed_nodes(nodes, nW, nb, g, gW, gb):
    return pl.pallas_call(
        _embed_nodes_body,
        out_shape=[
            jax.ShapeDtypeStruct((N_NODES, LATENT), jnp.float32),
            jax.ShapeDtypeStruct((1, LATENT), jnp.float32),
        ],
    )(nodes, nW, nb.reshape(1, LATENT), g, gW, gb.reshape(1, LATENT))


def _prep_body(n_ref, g_ref, ws_ref, wr_ref, wg_ref, b0_ref,
               ps_ref, pr_ref, ce_ref):
    n = n_ref[...]
    ps_ref[...] = jnp.dot(n, ws_ref[...], preferred_element_type=jnp.float32)
    pr_ref[...] = jnp.dot(n, wr_ref[...], preferred_element_type=jnp.float32)
    ce_ref[...] = jnp.dot(g_ref[...], wg_ref[...],
                          preferred_element_type=jnp.float32) + b0_ref[...]


def _prep(nodes, g, Ws, Wr, Wg, b0):
    return pl.pallas_call(
        _prep_body,
        out_shape=[
            jax.ShapeDtypeStruct((N_NODES, LATENT), jnp.float32),
            jax.ShapeDtypeStruct((N_NODES, LATENT), jnp.float32),
            jax.ShapeDtypeStruct((1, LATENT), jnp.float32),
        ],
    )(nodes, g, Ws, Wr, Wg, b0.reshape(1, LATENT))


def _edge_body(e_ref, gs_ref, gr_ref, we_ref, w1_ref, ce_ref, b1_ref,
               out_ref, agg_ref):
    h = jnp.dot(e_ref[...], we_ref[...], preferred_element_type=jnp.float32)
    h = h + gs_ref[...] + gr_ref[...] + ce_ref[...]
    h = _leaky(h)
    o = jnp.dot(h, w1_ref[...], preferred_element_type=jnp.float32) + b1_ref[...]
    out_ref[...] = o

    @pl.when(pl.program_id(0) == 0)
    def _():
        agg_ref[...] = jnp.zeros_like(agg_ref)

    agg_ref[...] += jnp.sum(o, axis=0, keepdims=True)


def _edge_update(edges, GS, GR, We, W1, ce, b1):
    return pl.pallas_call(
        _edge_body,
        grid=(NEB,),
        in_specs=[
            pl.BlockSpec((BE, LATENT), lambda i: (i, 0)),
            pl.BlockSpec((BE, LATENT), lambda i: (i, 0)),
            pl.BlockSpec((BE, LATENT), lambda i: (i, 0)),
            pl.BlockSpec((LATENT, LATENT), lambda i: (0, 0)),
            pl.BlockSpec((LATENT, LATENT), lambda i: (0, 0)),
            pl.BlockSpec((1, LATENT), lambda i: (0, 0)),
            pl.BlockSpec((1, LATENT), lambda i: (0, 0)),
        ],
        out_specs=[
            pl.BlockSpec((BE, LATENT), lambda i: (i, 0)),
            pl.BlockSpec((1, LATENT), lambda i: (0, 0)),
        ],
        out_shape=[
            jax.ShapeDtypeStruct((N_EDGES, LATENT), jnp.float32),
            jax.ShapeDtypeStruct((1, LATENT), jnp.float32),
        ],
    )(edges, GS, GR, We, W1, ce, b1.reshape(1, LATENT))


def _node_body(n_ref, s_ref, r_ref, wn_ref, ws_ref, wr_ref, cn_ref,
               w1_ref, b1_ref,
               ea_ref, g_ref, wa_ref, wb_ref, wc_ref, bg_ref,
               wg1_ref, bg1_ref,
               out_ref, na_ref, go_ref, nblocks):
    h = (jnp.dot(n_ref[...], wn_ref[...], preferred_element_type=jnp.float32)
         + jnp.dot(s_ref[...], ws_ref[...], preferred_element_type=jnp.float32)
         + jnp.dot(r_ref[...], wr_ref[...], preferred_element_type=jnp.float32)
         + cn_ref[...])
    h = _leaky(h)
    o = jnp.dot(h, w1_ref[...], preferred_element_type=jnp.float32) + b1_ref[...]
    out_ref[...] = o

    @pl.when(pl.program_id(0) == 0)
    def _():
        na_ref[...] = jnp.zeros_like(na_ref)

    na_ref[...] += jnp.sum(o, axis=0, keepdims=True)

    @pl.when(pl.program_id(0) == nblocks - 1)
    def _():
        hg = (jnp.dot(na_ref[...], wa_ref[...], preferred_element_type=jnp.float32)
              + jnp.dot(ea_ref[...], wb_ref[...], preferred_element_type=jnp.float32)
              + jnp.dot(g_ref[...], wc_ref[...], preferred_element_type=jnp.float32)
              + bg_ref[...])
        hg = _leaky(hg)
        go_ref[...] = jnp.dot(hg, wg1_ref[...],
                              preferred_element_type=jnp.float32) + bg1_ref[...]


def _node_update(nodes, sent, recv, Wn, Wse, Wre, cn, W1, b1,
                 ea, g, Wa, Wb, Wc, bg0, Wg1, bg1):
    BN = 2000
    nb = N_NODES // BN
    full = lambda i: (0, 0)
    return pl.pallas_call(
        functools.partial(_node_body, nblocks=nb),
        grid=(nb,),
        in_specs=[
            pl.BlockSpec((BN, LATENT), lambda i: (i, 0)),
            pl.BlockSpec((BN, LATENT), lambda i: (i, 0)),
            pl.BlockSpec((BN, LATENT), lambda i: (i, 0)),
            pl.BlockSpec((LATENT, LATENT), full),
            pl.BlockSpec((LATENT, LATENT), full),
            pl.BlockSpec((LATENT, LATENT), full),
            pl.BlockSpec((1, LATENT), full),
            pl.BlockSpec((LATENT, LATENT), full),
            pl.BlockSpec((1, LATENT), full),
            pl.BlockSpec((1, LATENT), full),
            pl.BlockSpec((1, LATENT), full),
            pl.BlockSpec((LATENT, LATENT), full),
            pl.BlockSpec((LATENT, LATENT), full),
            pl.BlockSpec((LATENT, LATENT), full),
            pl.BlockSpec((1, LATENT), full),
            pl.BlockSpec((LATENT, LATENT), full),
            pl.BlockSpec((1, LATENT), full),
        ],
        out_specs=[
            pl.BlockSpec((BN, LATENT), lambda i: (i, 0)),
            pl.BlockSpec((1, LATENT), full),
            pl.BlockSpec((1, LATENT), full),
        ],
        out_shape=[
            jax.ShapeDtypeStruct((N_NODES, LATENT), jnp.float32),
            jax.ShapeDtypeStruct((1, LATENT), jnp.float32),
            jax.ShapeDtypeStruct((1, LATENT), jnp.float32),
        ],
    )(nodes, sent, recv, Wn, Wse, Wre, cn, W1, b1.reshape(1, LATENT),
      ea, g, Wa, Wb, Wc, bg0.reshape(1, LATENT), Wg1, bg1.reshape(1, LATENT))


def _gather2(ps, pr, senders, receivers):
    return jnp.take(ps, senders, axis=0), jnp.take(pr, receivers, axis=0)


def _segsum2(edges, senders, receivers):
    sent = jax.ops.segment_sum(edges, senders, num_segments=N_NODES)
    recv = jax.ops.segment_sum(edges, receivers, num_segments=N_NODES)
    return sent, recv


def kernel(nodes, edges, globals_, senders, receivers,
           emb_node_W, emb_node_b, emb_edge_W, emb_edge_b,
           emb_global_W, emb_global_b,
           edge0_W0, edge0_b0, edge0_W1, edge0_b1,
           node0_W0, node0_b0, node0_W1, node0_b1,
           glob0_W0, glob0_b0, glob0_W1, glob0_b1,
           edge1_W0, edge1_b0, edge1_W1, edge1_b1,
           node1_W0, node1_b0, node1_W1, node1_b1,
           glob1_W0, glob1_b0, glob1_W1, glob1_b1):
    p = locals()
    edges_l = _embed_edges(edges, emb_edge_W, emb_edge_b)
    nodes_l, g_l = _embed_nodes(nodes, emb_node_W, emb_node_b,
                                globals_, emb_global_W, emb_global_b)
    for s in range(2):
        W0 = p[f'edge{s}_W0']
        We, Ws, Wr, Wg = (W0[i * LATENT:(i + 1) * LATENT] for i in range(4))
        ps, pr, ce = _prep(nodes_l, g_l, Ws, Wr, Wg, p[f'edge{s}_b0'])
        GS, GR = _gather2(ps, pr, senders, receivers)
        edges_l, ea = _edge_update(edges_l, GS, GR, We,
                                   p[f'edge{s}_W1'], ce, p[f'edge{s}_b1'])
        sent, recv = _segsum2(edges_l, senders, receivers)
        W0n = p[f'node{s}_W0']
        Wn, Wse, Wre, Wgn = (W0n[i * LATENT:(i + 1) * LATENT] for i in range(4))
        cn = jnp.dot(g_l, Wgn) + p[f'node{s}_b0'].reshape(1, LATENT)
        W0g = p[f'glob{s}_W0']
        Wa, Wb, Wc = (W0g[i * LATENT:(i + 1) * LATENT] for i in range(3))
        nodes_l, na, g_l = _node_update(
            nodes_l, sent, recv, Wn, Wse, Wre, cn,
            p[f'node{s}_W1'], p[f'node{s}_b1'],
            ea, g_l, Wa, Wb, Wc, p[f'glob{s}_b0'],
            p[f'glob{s}_W1'], p[f'glob{s}_b1'])
    return nodes_l, edges_l, g_l


# speedup vs baseline: 4.6005x; 4.6005x over previous
import functools

import jax
import jax.numpy as jnp
from jax import lax
from jax.experimental import pallas as pl
from jax.experimental.pallas import tpu as pltpu
from jax.experimental.pallas import tpu_sc as plsc

N_NODES = 10000
N_EDGES = 320000
LATENT = 128
BE = 2000
NEB = N_EDGES // BE
NSLOPE = 0.01


def _leaky(x):
    return jnp.where(x >= 0, x, NSLOPE * x)


def _embed_edges_body(e_ref, w_ref, b_ref, o_ref):
    o_ref[...] = jnp.dot(e_ref[...], w_ref[...],
                         preferred_element_type=jnp.float32) + b_ref[...]


def _embed_edges(edges, W, b):
    return pl.pallas_call(
        _embed_edges_body,
        grid=(NEB,),
        in_specs=[
            pl.BlockSpec((BE, 16), lambda i: (i, 0)),
            pl.BlockSpec((16, LATENT), lambda i: (0, 0)),
            pl.BlockSpec((1, LATENT), lambda i: (0, 0)),
        ],
        out_specs=pl.BlockSpec((BE, LATENT), lambda i: (i, 0)),
        out_shape=jax.ShapeDtypeStruct((N_EDGES, LATENT), jnp.float32),
    )(edges, W, b.reshape(1, LATENT))


def _embed_nodes_body(n_ref, nw_ref, nb_ref, g_ref, gw_ref, gb_ref,
                      no_ref, go_ref):
    no_ref[...] = jnp.dot(n_ref[...], nw_ref[...],
                          preferred_element_type=jnp.float32) + nb_ref[...]
    go_ref[...] = jnp.dot(g_ref[...], gw_ref[...],
                          preferred_element_type=jnp.float32) + gb_ref[...]


def _embed_nodes(nodes, nW, nb, g, gW, gb):
    return pl.pallas_call(
        _embed_nodes_body,
        out_shape=[
            jax.ShapeDtypeStruct((N_NODES, LATENT), jnp.float32),
            jax.ShapeDtypeStruct((1, LATENT), jnp.float32),
        ],
    )(nodes, nW, nb.reshape(1, LATENT), g, gW, gb.reshape(1, LATENT))


def _prep_body(n_ref, g_ref, ws_ref, wr_ref, wg_ref, b0_ref,
               ps_ref, pr_ref, ce_ref):
    n = n_ref[...]
    ps_ref[...] = jnp.dot(n, ws_ref[...], preferred_element_type=jnp.float32)
    pr_ref[...] = jnp.dot(n, wr_ref[...], preferred_element_type=jnp.float32)
    ce_ref[...] = jnp.dot(g_ref[...], wg_ref[...],
                          preferred_element_type=jnp.float32) + b0_ref[...]


def _prep(nodes, g, Ws, Wr, Wg, b0):
    return pl.pallas_call(
        _prep_body,
        out_shape=[
            jax.ShapeDtypeStruct((N_NODES, LATENT), jnp.float32),
            jax.ShapeDtypeStruct((N_NODES, LATENT), jnp.float32),
            jax.ShapeDtypeStruct((1, LATENT), jnp.float32),
        ],
    )(nodes, g, Ws, Wr, Wg, b0.reshape(1, LATENT))


def _edge_body(e_ref, gs_ref, gr_ref, we_ref, w1_ref, ce_ref, b1_ref,
               out_ref, agg_ref):
    h = jnp.dot(e_ref[...], we_ref[...], preferred_element_type=jnp.float32)
    h = h + gs_ref[...] + gr_ref[...] + ce_ref[...]
    h = _leaky(h)
    o = jnp.dot(h, w1_ref[...], preferred_element_type=jnp.float32) + b1_ref[...]
    out_ref[...] = o

    @pl.when(pl.program_id(0) == 0)
    def _():
        agg_ref[...] = jnp.zeros_like(agg_ref)

    agg_ref[...] += jnp.sum(o, axis=0, keepdims=True)


def _edge_update(edges, GS, GR, We, W1, ce, b1):
    return pl.pallas_call(
        _edge_body,
        grid=(NEB,),
        in_specs=[
            pl.BlockSpec((BE, LATENT), lambda i: (i, 0)),
            pl.BlockSpec((BE, LATENT), lambda i: (i, 0)),
            pl.BlockSpec((BE, LATENT), lambda i: (i, 0)),
            pl.BlockSpec((LATENT, LATENT), lambda i: (0, 0)),
            pl.BlockSpec((LATENT, LATENT), lambda i: (0, 0)),
            pl.BlockSpec((1, LATENT), lambda i: (0, 0)),
            pl.BlockSpec((1, LATENT), lambda i: (0, 0)),
        ],
        out_specs=[
            pl.BlockSpec((BE, LATENT), lambda i: (i, 0)),
            pl.BlockSpec((1, LATENT), lambda i: (0, 0)),
        ],
        out_shape=[
            jax.ShapeDtypeStruct((N_EDGES, LATENT), jnp.float32),
            jax.ShapeDtypeStruct((1, LATENT), jnp.float32),
        ],
    )(edges, GS, GR, We, W1, ce, b1.reshape(1, LATENT))


def _node_body(n_ref, s_ref, r_ref, wn_ref, ws_ref, wr_ref, cn_ref,
               w1_ref, b1_ref,
               ea_ref, g_ref, wa_ref, wb_ref, wc_ref, bg_ref,
               wg1_ref, bg1_ref,
               out_ref, na_ref, go_ref, nblocks):
    h = (jnp.dot(n_ref[...], wn_ref[...], preferred_element_type=jnp.float32)
         + jnp.dot(s_ref[...], ws_ref[...], preferred_element_type=jnp.float32)
         + jnp.dot(r_ref[...], wr_ref[...], preferred_element_type=jnp.float32)
         + cn_ref[...])
    h = _leaky(h)
    o = jnp.dot(h, w1_ref[...], preferred_element_type=jnp.float32) + b1_ref[...]
    out_ref[...] = o

    @pl.when(pl.program_id(0) == 0)
    def _():
        na_ref[...] = jnp.zeros_like(na_ref)

    na_ref[...] += jnp.sum(o, axis=0, keepdims=True)

    @pl.when(pl.program_id(0) == nblocks - 1)
    def _():
        hg = (jnp.dot(na_ref[...], wa_ref[...], preferred_element_type=jnp.float32)
              + jnp.dot(ea_ref[...], wb_ref[...], preferred_element_type=jnp.float32)
              + jnp.dot(g_ref[...], wc_ref[...], preferred_element_type=jnp.float32)
              + bg_ref[...])
        hg = _leaky(hg)
        go_ref[...] = jnp.dot(hg, wg1_ref[...],
                              preferred_element_type=jnp.float32) + bg1_ref[...]


def _node_update(nodes, sent, recv, Wn, Wse, Wre, cn, W1, b1,
                 ea, g, Wa, Wb, Wc, bg0, Wg1, bg1):
    BN = 2000
    nb = N_NODES // BN
    full = lambda i: (0, 0)
    return pl.pallas_call(
        functools.partial(_node_body, nblocks=nb),
        grid=(nb,),
        in_specs=[
            pl.BlockSpec((BN, LATENT), lambda i: (i, 0)),
            pl.BlockSpec((BN, LATENT), lambda i: (i, 0)),
            pl.BlockSpec((BN, LATENT), lambda i: (i, 0)),
            pl.BlockSpec((LATENT, LATENT), full),
            pl.BlockSpec((LATENT, LATENT), full),
            pl.BlockSpec((LATENT, LATENT), full),
            pl.BlockSpec((1, LATENT), full),
            pl.BlockSpec((LATENT, LATENT), full),
            pl.BlockSpec((1, LATENT), full),
            pl.BlockSpec((1, LATENT), full),
            pl.BlockSpec((1, LATENT), full),
            pl.BlockSpec((LATENT, LATENT), full),
            pl.BlockSpec((LATENT, LATENT), full),
            pl.BlockSpec((LATENT, LATENT), full),
            pl.BlockSpec((1, LATENT), full),
            pl.BlockSpec((LATENT, LATENT), full),
            pl.BlockSpec((1, LATENT), full),
        ],
        out_specs=[
            pl.BlockSpec((BN, LATENT), lambda i: (i, 0)),
            pl.BlockSpec((1, LATENT), full),
            pl.BlockSpec((1, LATENT), full),
        ],
        out_shape=[
            jax.ShapeDtypeStruct((N_NODES, LATENT), jnp.float32),
            jax.ShapeDtypeStruct((1, LATENT), jnp.float32),
            jax.ShapeDtypeStruct((1, LATENT), jnp.float32),
        ],
    )(nodes, sent, recv, Wn, Wse, Wre, cn, W1, b1.reshape(1, LATENT),
      ea, g, Wa, Wb, Wc, bg0.reshape(1, LATENT), Wg1, bg1.reshape(1, LATENT))


_SC_MESH = plsc.VectorSubcoreMesh(core_axis_name="c", subcore_axis_name="s")
GW = 128
SW = 128
NSUB = 16
NPS = 632
N_NODES_PAD = NPS * NSUB


def _sc_gather_body(ps_hbm, pr_hbm, snd_hbm, rcv_hbm, gs_hbm, gr_hbm):
    c = lax.axis_index("c")

    def mk(tab_hbm):
        def body(i_vmem, o_vmem):
            pltpu.sync_copy(tab_hbm.at[i_vmem.at[0]], o_vmem)
        return pltpu.emit_pipeline(
            body,
            grid=(N_EDGES // GW,),
            in_specs=[pl.BlockSpec((1, GW), lambda i: (0, i))],
            out_specs=[pl.BlockSpec((GW, LATENT), lambda i: (i, 0))],
            core_axis_name="s",
            dimension_semantics=(pltpu.PARALLEL,),
        )

    @pl.when(c == 0)
    def _():
        mk(ps_hbm)(snd_hbm, gs_hbm)

    @pl.when(c == 1)
    def _():
        mk(pr_hbm)(rcv_hbm, gr_hbm)


def _gather2(ps, pr, senders2d, receivers2d):
    f = pl.kernel(
        _sc_gather_body,
        out_type=[
            jax.ShapeDtypeStruct((N_EDGES, LATENT), jnp.float32),
            jax.ShapeDtypeStruct((N_EDGES, LATENT), jnp.float32),
        ],
        mesh=_SC_MESH,
    )
    return f(ps, pr, senders2d, receivers2d)


def _sc_segsum_body(edges_hbm, snd_hbm, rcv_hbm, zeros_hbm,
                    sent_hbm, recv_hbm, acc):
    c = lax.axis_index("c")
    s = lax.axis_index("s")

    pltpu.sync_copy(zeros_hbm, acc.at[pl.ds(s * NPS, NPS)])
    plsc.subcore_barrier()

    def body(e_vmem, i_vmem):
        pltpu.sync_copy(e_vmem, acc.at[i_vmem.at[0]], add=True)

    pipe = pltpu.emit_pipeline(
        body,
        grid=(N_EDGES // SW,),
        in_specs=[
            pl.BlockSpec((SW, LATENT), lambda i: (i, 0)),
            pl.BlockSpec((1, SW), lambda i: (0, i)),
        ],
        out_specs=[],
        core_axis_name="s",
        dimension_semantics=(pltpu.PARALLEL,),
    )

    @pl.when(c == 0)
    def _():
        pipe(edges_hbm, snd_hbm)

    @pl.when(c == 1)
    def _():
        pipe(edges_hbm, rcv_hbm)

    plsc.subcore_barrier()
    last = N_NODES - 15 * NPS

    @pl.when(c == 0)
    def _():
        @pl.when(s < 15)
        def _():
            rows = pl.ds(s * NPS, NPS)
            pltpu.sync_copy(acc.at[rows], sent_hbm.at[rows])

        @pl.when(s == 15)
        def _():
            rows = pl.ds(15 * NPS, last)
            pltpu.sync_copy(acc.at[rows], sent_hbm.at[rows])

    @pl.when(c == 1)
    def _():
        @pl.when(s < 15)
        def _():
            rows = pl.ds(s * NPS, NPS)
            pltpu.sync_copy(acc.at[rows], recv_hbm.at[rows])

        @pl.when(s == 15)
        def _():
            rows = pl.ds(15 * NPS, last)
            pltpu.sync_copy(acc.at[rows], recv_hbm.at[rows])


def _segsum2(edges, senders2d, receivers2d):
    zeros = jnp.zeros((NPS, LATENT), jnp.float32)
    f = pl.kernel(
        _sc_segsum_body,
        out_type=[
            jax.ShapeDtypeStruct((N_NODES, LATENT), jnp.float32),
            jax.ShapeDtypeStruct((N_NODES, LATENT), jnp.float32),
        ],
        mesh=_SC_MESH,
        scratch_types=[
            pltpu.VMEM_SHARED((N_NODES_PAD, LATENT), jnp.float32),
        ],
    )
    return f(edges, senders2d, receivers2d, zeros)


def kernel(nodes, edges, globals_, senders, receivers,
           emb_node_W, emb_node_b, emb_edge_W, emb_edge_b,
           emb_global_W, emb_global_b,
           edge0_W0, edge0_b0, edge0_W1, edge0_b1,
           node0_W0, node0_b0, node0_W1, node0_b1,
           glob0_W0, glob0_b0, glob0_W1, glob0_b1,
           edge1_W0, edge1_b0, edge1_W1, edge1_b1,
           node1_W0, node1_b0, node1_W1, node1_b1,
           glob1_W0, glob1_b0, glob1_W1, glob1_b1):
    p = locals()
    snd2d = senders.reshape(1, N_EDGES)
    rcv2d = receivers.reshape(1, N_EDGES)
    edges_l = _embed_edges(edges, emb_edge_W, emb_edge_b)
    nodes_l, g_l = _embed_nodes(nodes, emb_node_W, emb_node_b,
                                globals_, emb_global_W, emb_global_b)
    for s in range(2):
        W0 = p[f'edge{s}_W0']
        We, Ws, Wr, Wg = (W0[i * LATENT:(i + 1) * LATENT] for i in range(4))
        ps, pr, ce = _prep(nodes_l, g_l, Ws, Wr, Wg, p[f'edge{s}_b0'])
        GS, GR = _gather2(ps, pr, snd2d, rcv2d)
        edges_l, ea = _edge_update(edges_l, GS, GR, We,
                                   p[f'edge{s}_W1'], ce, p[f'edge{s}_b1'])
        sent, recv = _segsum2(edges_l, snd2d, rcv2d)
        W0n = p[f'node{s}_W0']
        Wn, Wse, Wre, Wgn = (W0n[i * LATENT:(i + 1) * LATENT] for i in range(4))
        cn = jnp.dot(g_l, Wgn) + p[f'node{s}_b0'].reshape(1, LATENT)
        W0g = p[f'glob{s}_W0']
        Wa, Wb, Wc = (W0g[i * LATENT:(i + 1) * LATENT] for i in range(3))
        nodes_l, na, g_l = _node_update(
            nodes_l, sent, recv, Wn, Wse, Wre, cn,
            p[f'node{s}_W1'], p[f'node{s}_b1'],
            ea, g_l, Wa, Wb, Wc, p[f'glob{s}_b0'],
            p[f'glob{s}_W1'], p[f'glob{s}_b1'])
    return nodes_l, edges_l, g_l
